# Initial kernel scaffold; baseline (speedup 1.0000x reference)
import jax
import jax.numpy as jnp
from jax.experimental import pallas as pl


def kernel(x, edge_index, edge_attr, params):
    raise NotImplementedError("write your pallas kernel here")


# speedup vs baseline: 2.0556x; 2.0556x over previous
import functools

import jax
import jax.numpy as jnp
from jax import lax
from jax.experimental import pallas as pl
from jax.experimental.pallas import tpu as pltpu
from jax.experimental.pallas import tpu_sc as plsc

N = 10000
E = 320000
D = 128
NW = 32
EPW = E // NW
NCH = EPW // 16
ACCW = 144
NPAD = 10240
ROWW = 192


def _prep_body(src_ref, vr_ref, vc_ref, vl_ref, cidx_ref):
    is_r = (jnp.abs(vr_ref[...]) > 0.01).astype(jnp.int32)
    is_c = (jnp.abs(vc_ref[...]) > 0.01).astype(jnp.int32)
    is_l = (jnp.abs(vl_ref[...]) > 0.01).astype(jnp.int32)
    combo = is_r + 2 * is_c + 4 * is_l
    cidx_ref[...] = combo * N + src_ref[...]


def _prep(src2, vr2, vc2, vl2):
    return pl.pallas_call(
        _prep_body,
        out_shape=jax.ShapeDtypeStruct(src2.shape, jnp.int32),
    )(src2, vr2, vc2, vl2)


def _tables_body(h_ref, wn_ref, wr_ref, wc_ref, wl_ref, w1a_ref, w1b_ref,
                 b1_ref, t8_ref, adst_ref):
    x_lin = jnp.dot(h_ref[...], wn_ref[...], preferred_element_type=jnp.float32)
    m_r = jnp.dot(x_lin, wr_ref[...], preferred_element_type=jnp.float32)
    m_c = jnp.dot(x_lin, wc_ref[...], preferred_element_type=jnp.float32)
    m_l = jnp.dot(x_lin, wl_ref[...], preferred_element_type=jnp.float32)
    a_src = jnp.dot(x_lin, w1b_ref[...], preferred_element_type=jnp.float32)
    a_dst = jnp.dot(x_lin, w1a_ref[...], preferred_element_type=jnp.float32)
    adst_ref[...] = a_dst + b1_ref[...]
    z = jnp.zeros_like(m_r)
    combos = (z, m_r, m_c, m_r + m_c, m_l, m_r + m_l, m_c + m_l,
              m_r + m_c + m_l)
    t8_ref[...] = jnp.stack(
        [jnp.concatenate([cmb, a_src], axis=1) for cmb in combos], axis=0)


def _tables(h, wn_t, wr_t, wc_t, wl_t, w1a_t, w1b_t, b1):
    blk = 1000
    grid = N // blk
    full = lambda shape: pl.BlockSpec(shape, lambda i: (0,) * len(shape))
    return pl.pallas_call(
        _tables_body,
        grid=(grid,),
        in_specs=[
            pl.BlockSpec((blk, D), lambda i: (i, 0)),
            full((D, D)), full((D, D)), full((D, D)), full((D, D)),
            full((D, 64)), full((D, 64)), full((1, 64)),
        ],
        out_specs=[
            pl.BlockSpec((8, blk, ROWW), lambda i: (0, i, 0)),
            pl.BlockSpec((blk, 64), lambda i: (i, 0)),
        ],
        out_shape=[
            jax.ShapeDtypeStruct((8, N, ROWW), jnp.float32),
            jax.ShapeDtypeStruct((N, 64), jnp.float32),
        ],
    )(h, wn_t, wr_t, wc_t, wl_t, w1a_t, w1b_t, b1)


def _sc_edge_body(t8_hbm, adst_hbm, cidx_hbm, dst_hbm, attr_hbm, w2_hbm,
                  b2_hbm, zeros_hbm, out_hbm,
                  cidx_v, dst_v, attr_c, w2_v, b2_v, rows_v, adst_v,
                  stage_v, hacc_v, aux_tr, acc_sh):
    c = lax.axis_index("c")
    s = lax.axis_index("s")
    wid = c * 16 + s

    pltpu.sync_copy(cidx_hbm.at[wid], cidx_v)
    pltpu.sync_copy(dst_hbm.at[wid], dst_v)
    pltpu.sync_copy(w2_hbm, w2_v)
    pltpu.sync_copy(b2_hbm, b2_v)

    rpw = NPAD // 16
    pltpu.sync_copy(zeros_hbm.at[pl.ds(s * rpw, rpw)],
                    acc_sh.at[pl.ds(s * rpw, rpw)])

    iota = lax.iota(jnp.int32, 16)
    zero16 = jnp.zeros((16,), jnp.float32)
    for e in range(16):
        aux_tr[pl.ds(e * 16, 16)] = zero16
    plsc.subcore_barrier()

    def chunk(k, carry):
        idx_row = cidx_v.at[k]
        dst_row = dst_v.at[k]
        pltpu.sync_copy(t8_hbm.at[idx_row], rows_v)
        pltpu.sync_copy(adst_hbm.at[dst_row], adst_v)
        pltpu.sync_copy(attr_hbm.at[wid, k], attr_c)

        for e in range(16):
            acc = zero16
            for q in range(4):
                a_s = rows_v[e, pl.ds(D + q * 16, 16)]
                a_d = adst_v[e, pl.ds(q * 16, 16)]
                hrelu = jnp.maximum(a_s + a_d, 0.0)
                acc = acc + hrelu * w2_v[pl.ds(q * 16, 16)]
            hacc_v[pl.ds(e * 16, 16)] = acc
        z = zero16
        for l in range(16):
            z = z + plsc.load_gather(hacc_v, [iota * 16 + l])
        z = z + b2_v[...]
        att = 1.0 / (1.0 + jnp.exp(-z))

        v_r = attr_c[pl.ds(0, 16)]
        v_c = attr_c[pl.ds(16, 16)]
        v_l = attr_c[pl.ds(32, 16)]
        a_r = jnp.where(jnp.abs(v_r) > 0.01, att, 0.0)
        a_c = jnp.where(jnp.abs(v_c) > 0.01, att, 0.0)
        a_l = jnp.where(jnp.abs(v_l) > 0.01, att, 0.0)
        for j, vec in enumerate((a_r * v_r, a_c * v_c, a_l * v_l,
                                 a_r, a_c, a_l)):
            plsc.store_scatter(aux_tr, [iota * 16 + j], vec)

        for e in range(16):
            att_e = att[e]
            for q in range(8):
                stage_v[e, pl.ds(q * 16, 16)] = (
                    rows_v[e, pl.ds(q * 16, 16)] * att_e)
            stage_v[e, pl.ds(D, 16)] = aux_tr[pl.ds(e * 16, 16)]

        pltpu.sync_copy(stage_v, acc_sh.at[dst_row], add=True)
        return carry

    lax.fori_loop(0, NCH, chunk, 0)
    plsc.subcore_barrier()

    pltpu.sync_copy(acc_sh.at[pl.ds(s * rpw, rpw)],
                    out_hbm.at[c, pl.ds(s * rpw, rpw)])


@functools.partial(jax.jit, static_argnums=())
def _sc_edge(t8_flat, adst_tab, cidx_r, dst_r, attr_r, w2, b2v, zeros):
    mesh = plsc.VectorSubcoreMesh(core_axis_name="c", subcore_axis_name="s")
    kern = pl.kernel(
        _sc_edge_body,
        out_type=jax.ShapeDtypeStruct((2, NPAD, ACCW), jnp.float32),
        mesh=mesh,
        compiler_params=pltpu.CompilerParams(use_tc_tiling_on_sc=False,
                                             needs_layout_passes=False),
        scratch_types=[
            pltpu.VMEM((NCH, 16), jnp.int32),
            pltpu.VMEM((NCH, 16), jnp.int32),
            pltpu.VMEM((48,), jnp.float32),
            pltpu.VMEM((64,), jnp.float32),
            pltpu.VMEM((16,), jnp.float32),
            pltpu.VMEM((16, ROWW), jnp.float32),
            pltpu.VMEM((16, 64), jnp.float32),
            pltpu.VMEM((16, ACCW), jnp.float32),
            pltpu.VMEM((256,), jnp.float32),
            pltpu.VMEM((256,), jnp.float32),
            pltpu.VMEM_SHARED((NPAD, ACCW), jnp.float32),
        ],
    )
    return kern(t8_flat, adst_tab, cidx_r, dst_r, attr_r, w2, b2v, zeros)


def _final_body(acc_ref, bmat_ref, bias_ref, g_ref, b_ref, hin_ref, out_ref,
                *, apply_relu):
    a = acc_ref[0] + acc_ref[1]
    out = a[:, :D] + jnp.dot(a[:, D:ACCW], bmat_ref[...],
                             preferred_element_type=jnp.float32)
    out = out + bias_ref[...]
    mu = jnp.mean(out, axis=1, keepdims=True)
    var = jnp.mean((out - mu) ** 2, axis=1, keepdims=True)
    out = (out - mu) * lax.rsqrt(var + 1e-5) * g_ref[...] + b_ref[...]
    if apply_relu:
        out = jnp.maximum(out, 0.0)
    out_ref[...] = out + hin_ref[...]


def _finalize(acc, bmat, bias, g, b, h_in, apply_relu):
    blk = 1000
    grid = N // blk
    full = lambda shape: pl.BlockSpec(shape, lambda i: (0,) * len(shape))
    return pl.pallas_call(
        functools.partial(_final_body, apply_relu=apply_relu),
        grid=(grid,),
        in_specs=[
            pl.BlockSpec((2, blk, ACCW), lambda i: (0, i, 0)),
            full((ACCW - D, D)), full((1, D)), full((1, D)), full((1, D)),
            pl.BlockSpec((blk, D), lambda i: (i, 0)),
        ],
        out_specs=pl.BlockSpec((blk, D), lambda i: (i, 0)),
        out_shape=jax.ShapeDtypeStruct((N, D), jnp.float32),
    )(acc, bmat, bias, g, b, h_in)


def kernel(x, edge_index, edge_attr, params):
    src = edge_index[0]
    dst = edge_index[1]
    vr = edge_attr[:, 0]
    vc = edge_attr[:, 1]
    vl = edge_attr[:, 2]

    cidx = _prep(src.reshape(2500, 128), vr.reshape(2500, 128),
                 vc.reshape(2500, 128), vl.reshape(2500, 128))
    cidx_r = cidx.reshape(NW, NCH, 16)
    dst_r = dst.reshape(NW, NCH, 16)
    attr_r = jnp.transpose(edge_attr.reshape(NW, NCH, 16, 3),
                           (0, 1, 3, 2)).reshape(NW, NCH, 48)
    zeros = jnp.zeros((NPAD, ACCW), jnp.float32)

    h = x
    nlayers = len(params)
    for i, p in enumerate(params):
        w1a_t = p['W_att1'][:, :D].T
        w1b_t = p['W_att1'][:, D:].T
        t8, adst_tab = _tables(
            h, p['W_node'].T, p['W_R'][:, :D].T, p['W_C'][:, :D].T,
            p['W_L'][:, :D].T, w1a_t, w1b_t, p['b_att1'].reshape(1, 64))
        w2 = p['W_att2'][0]
        b2v = jnp.full((16,), p['b_att2'][0], jnp.float32)
        acc = _sc_edge(t8.reshape(8 * N, ROWW), adst_tab, cidx_r, dst_r,
                       attr_r, w2, b2v, zeros)
        bmat = jnp.concatenate([
            p['W_R'][:, D:D + 1].T, p['W_C'][:, D:D + 1].T,
            p['W_L'][:, D:D + 1].T, p['b_R'].reshape(1, D),
            p['b_C'].reshape(1, D), p['b_L'].reshape(1, D),
            jnp.zeros((ACCW - D - 6, D), jnp.float32)], axis=0)
        h = _finalize(acc, bmat, p['bias'].reshape(1, D),
                      p['ln_g'].reshape(1, D), p['ln_b'].reshape(1, D),
                      h, apply_relu=(i < nlayers - 1))
    return h


# speedup vs baseline: 5.9041x; 2.8722x over previous
import functools

import jax
import jax.numpy as jnp
from jax import lax
from jax.experimental import pallas as pl
from jax.experimental.pallas import tpu as pltpu
from jax.experimental.pallas import tpu_sc as plsc

N = 10000
E = 320000
D = 128
NW = 32
EPW = E // NW
NCH = EPW // 16
ACCW = 144
NPAD = 10240
ROWW = 192


def _prep_body(src_ref, vr_ref, vc_ref, vl_ref, cidx_ref):
    is_r = (jnp.abs(vr_ref[...]) > 0.01).astype(jnp.int32)
    is_c = (jnp.abs(vc_ref[...]) > 0.01).astype(jnp.int32)
    is_l = (jnp.abs(vl_ref[...]) > 0.01).astype(jnp.int32)
    combo = is_r + 2 * is_c + 4 * is_l
    cidx_ref[...] = combo * N + src_ref[...]


def _prep(src2, vr2, vc2, vl2):
    return pl.pallas_call(
        _prep_body,
        out_shape=jax.ShapeDtypeStruct(src2.shape, jnp.int32),
    )(src2, vr2, vc2, vl2)


def _tables_body(h_ref, wn_ref, wr_ref, wc_ref, wl_ref, w1a_ref, w1b_ref,
                 b1_ref, t8_ref, adst_ref):
    x_lin = jnp.dot(h_ref[...], wn_ref[...], preferred_element_type=jnp.float32)
    m_r = jnp.dot(x_lin, wr_ref[...], preferred_element_type=jnp.float32)
    m_c = jnp.dot(x_lin, wc_ref[...], preferred_element_type=jnp.float32)
    m_l = jnp.dot(x_lin, wl_ref[...], preferred_element_type=jnp.float32)
    a_src = jnp.dot(x_lin, w1b_ref[...], preferred_element_type=jnp.float32)
    a_dst = jnp.dot(x_lin, w1a_ref[...], preferred_element_type=jnp.float32)
    adst_ref[...] = a_dst + b1_ref[...]
    z = jnp.zeros_like(m_r)
    combos = (z, m_r, m_c, m_r + m_c, m_l, m_r + m_l, m_c + m_l,
              m_r + m_c + m_l)
    t8_ref[...] = jnp.stack(
        [jnp.concatenate([cmb, a_src], axis=1) for cmb in combos], axis=0)


def _tables(h, wn_t, wr_t, wc_t, wl_t, w1a_t, w1b_t, b1):
    blk = 1000
    grid = N // blk
    full = lambda shape: pl.BlockSpec(shape, lambda i: (0,) * len(shape))
    return pl.pallas_call(
        _tables_body,
        grid=(grid,),
        in_specs=[
            pl.BlockSpec((blk, D), lambda i: (i, 0)),
            full((D, D)), full((D, D)), full((D, D)), full((D, D)),
            full((D, 64)), full((D, 64)), full((1, 64)),
        ],
        out_specs=[
            pl.BlockSpec((8, blk, ROWW), lambda i: (0, i, 0)),
            pl.BlockSpec((blk, 64), lambda i: (i, 0)),
        ],
        out_shape=[
            jax.ShapeDtypeStruct((8, N, ROWW), jnp.float32),
            jax.ShapeDtypeStruct((N, 64), jnp.float32),
        ],
    )(h, wn_t, wr_t, wc_t, wl_t, w1a_t, w1b_t, b1)


def _sc_edge_body(t8_hbm, adst_hbm, cidx_hbm, dst_hbm, attr_hbm, w2_hbm,
                  b2_hbm, zeros_hbm, out_hbm,
                  cidx_v, dst_v, w2_v, b2_v, rows0, rows1, adst0, adst1,
                  attr0, attr1, stage0, stage1, hacc_v, aux_tr, acc_sh,
                  gsem0, gsem1, ssem0, ssem1):
    c = lax.axis_index("c")
    s = lax.axis_index("s")
    wid = c * 16 + s

    pltpu.sync_copy(cidx_hbm.at[wid], cidx_v)
    pltpu.sync_copy(dst_hbm.at[wid], dst_v)
    pltpu.sync_copy(w2_hbm, w2_v)
    pltpu.sync_copy(b2_hbm, b2_v)

    rpw = NPAD // 16
    pltpu.sync_copy(zeros_hbm.at[pl.ds(s * rpw, rpw)],
                    acc_sh.at[pl.ds(s * rpw, rpw)])

    iota = lax.iota(jnp.int32, 16)
    zero16 = jnp.zeros((16,), jnp.float32)
    for e in range(16):
        aux_tr[pl.ds(e * 16, 16)] = zero16
    plsc.subcore_barrier()

    bufs = ((rows0, adst0, attr0, stage0, gsem0, ssem0),
            (rows1, adst1, attr1, stage1, gsem1, ssem1))

    def fire_gathers(k, b):
        rows_v, adst_v, attr_c, _, gsem, _ = bufs[b]
        pltpu.async_copy(t8_hbm.at[cidx_v.at[k]], rows_v, gsem)
        pltpu.async_copy(adst_hbm.at[dst_v.at[k]], adst_v, gsem)
        pltpu.async_copy(attr_hbm.at[wid, k], attr_c, gsem)

    def process(k, b):
        rows_v, adst_v, attr_c, stage_v, gsem, ssem = bufs[b]
        dst_row = dst_v.at[k]
        pltpu.make_async_copy(t8_hbm.at[cidx_v.at[k]], rows_v, gsem).wait()
        pltpu.make_async_copy(adst_hbm.at[dst_v.at[k]], adst_v, gsem).wait()
        pltpu.make_async_copy(attr_hbm.at[wid, k], attr_c, gsem).wait()

        w2q = [w2_v[pl.ds(q * 16, 16)] for q in range(4)]
        for e in range(16):
            acc = zero16
            for q in range(4):
                a_s = rows_v[e, pl.ds(D + q * 16, 16)]
                a_d = adst_v[e, pl.ds(q * 16, 16)]
                hrelu = jnp.maximum(a_s + a_d, 0.0)
                acc = acc + hrelu * w2q[q]
            hacc_v[pl.ds(e * 16, 16)] = acc
        z = b2_v[...]
        for l in range(16):
            z = z + plsc.load_gather(hacc_v, [iota * 16 + l])
        att = 1.0 / (1.0 + jnp.exp(-z))

        v_r = attr_c[pl.ds(0, 16)]
        v_c = attr_c[pl.ds(16, 16)]
        v_l = attr_c[pl.ds(32, 16)]
        a_r = jnp.where(jnp.abs(v_r) > 0.01, att, 0.0)
        a_c = jnp.where(jnp.abs(v_c) > 0.01, att, 0.0)
        a_l = jnp.where(jnp.abs(v_l) > 0.01, att, 0.0)
        for j, vec in enumerate((a_r * v_r, a_c * v_c, a_l * v_l,
                                 a_r, a_c, a_l)):
            plsc.store_scatter(aux_tr, [iota * 16 + j], vec)

        if isinstance(k, int):
            if k >= 2:
                pltpu.make_async_copy(stage_v, acc_sh.at[dst_row], ssem).wait()
        else:

            @pl.when(k >= 2)
            def _():
                pltpu.make_async_copy(stage_v, acc_sh.at[dst_row], ssem).wait()

        for e in range(16):
            att_e = att[e]
            for q in range(8):
                stage_v[e, pl.ds(q * 16, 16)] = (
                    rows_v[e, pl.ds(q * 16, 16)] * att_e)
            stage_v[e, pl.ds(D, 16)] = aux_tr[pl.ds(e * 16, 16)]

        pltpu.async_copy(stage_v, acc_sh.at[dst_row], ssem, add=True)

    fire_gathers(0, 0)
    fire_gathers(1, 1)

    def pair(g, carry):
        k0 = g * 2
        process(k0, 0)
        fire_gathers(k0 + 2, 0)
        process(k0 + 1, 1)

        @pl.when(k0 + 3 < NCH)
        def _():
            fire_gathers(k0 + 3, 1)

        return carry

    lax.fori_loop(0, (NCH - 1) // 2, pair, 0)
    process(NCH - 1, 0)

    pltpu.make_async_copy(stage1, acc_sh.at[dst_v.at[NCH - 2]], ssem1).wait()
    pltpu.make_async_copy(stage0, acc_sh.at[dst_v.at[NCH - 1]], ssem0).wait()
    plsc.subcore_barrier()

    pltpu.sync_copy(acc_sh.at[pl.ds(s * rpw, rpw)],
                    out_hbm.at[c, pl.ds(s * rpw, rpw)])


@functools.partial(jax.jit, static_argnums=())
def _sc_edge(t8_flat, adst_tab, cidx_r, dst_r, attr_r, w2, b2v, zeros):
    mesh = plsc.VectorSubcoreMesh(core_axis_name="c", subcore_axis_name="s")
    kern = pl.kernel(
        _sc_edge_body,
        out_type=jax.ShapeDtypeStruct((2, NPAD, ACCW), jnp.float32),
        mesh=mesh,
        compiler_params=pltpu.CompilerParams(use_tc_tiling_on_sc=False,
                                             needs_layout_passes=False),
        scratch_types=[
            pltpu.VMEM((NCH, 16), jnp.int32),
            pltpu.VMEM((NCH, 16), jnp.int32),
            pltpu.VMEM((64,), jnp.float32),
            pltpu.VMEM((16,), jnp.float32),
            pltpu.VMEM((16, ROWW), jnp.float32),
            pltpu.VMEM((16, ROWW), jnp.float32),
            pltpu.VMEM((16, 64), jnp.float32),
            pltpu.VMEM((16, 64), jnp.float32),
            pltpu.VMEM((48,), jnp.float32),
            pltpu.VMEM((48,), jnp.float32),
            pltpu.VMEM((16, ACCW), jnp.float32),
            pltpu.VMEM((16, ACCW), jnp.float32),
            pltpu.VMEM((256,), jnp.float32),
            pltpu.VMEM((256,), jnp.float32),
            pltpu.VMEM_SHARED((NPAD, ACCW), jnp.float32),
            pltpu.SemaphoreType.DMA,
            pltpu.SemaphoreType.DMA,
            pltpu.SemaphoreType.DMA,
            pltpu.SemaphoreType.DMA,
        ],
    )
    return kern(t8_flat, adst_tab, cidx_r, dst_r, attr_r, w2, b2v, zeros)


def _final_body(acc_ref, bmat_ref, bias_ref, g_ref, b_ref, hin_ref, out_ref,
                *, apply_relu):
    a = acc_ref[0] + acc_ref[1]
    out = a[:, :D] + jnp.dot(a[:, D:ACCW], bmat_ref[...],
                             preferred_element_type=jnp.float32)
    out = out + bias_ref[...]
    mu = jnp.mean(out, axis=1, keepdims=True)
    var = jnp.mean((out - mu) ** 2, axis=1, keepdims=True)
    out = (out - mu) * lax.rsqrt(var + 1e-5) * g_ref[...] + b_ref[...]
    if apply_relu:
        out = jnp.maximum(out, 0.0)
    out_ref[...] = out + hin_ref[...]


def _finalize(acc, bmat, bias, g, b, h_in, apply_relu):
    blk = 1000
    grid = N // blk
    full = lambda shape: pl.BlockSpec(shape, lambda i: (0,) * len(shape))
    return pl.pallas_call(
        functools.partial(_final_body, apply_relu=apply_relu),
        grid=(grid,),
        in_specs=[
            pl.BlockSpec((2, blk, ACCW), lambda i: (0, i, 0)),
            full((ACCW - D, D)), full((1, D)), full((1, D)), full((1, D)),
            pl.BlockSpec((blk, D), lambda i: (i, 0)),
        ],
        out_specs=pl.BlockSpec((blk, D), lambda i: (i, 0)),
        out_shape=jax.ShapeDtypeStruct((N, D), jnp.float32),
    )(acc, bmat, bias, g, b, h_in)


def kernel(x, edge_index, edge_attr, params):
    src = edge_index[0]
    dst = edge_index[1]
    vr = edge_attr[:, 0]
    vc = edge_attr[:, 1]
    vl = edge_attr[:, 2]

    cidx = _prep(src.reshape(2500, 128), vr.reshape(2500, 128),
                 vc.reshape(2500, 128), vl.reshape(2500, 128))
    cidx_r = cidx.reshape(NW, NCH, 16)
    dst_r = dst.reshape(NW, NCH, 16)
    attr_r = jnp.transpose(edge_attr.reshape(NW, NCH, 16, 3),
                           (0, 1, 3, 2)).reshape(NW, NCH, 48)
    zeros = jnp.zeros((NPAD, ACCW), jnp.float32)

    h = x
    nlayers = len(params)
    for i, p in enumerate(params):
        w1a_t = p['W_att1'][:, :D].T
        w1b_t = p['W_att1'][:, D:].T
        t8, adst_tab = _tables(
            h, p['W_node'].T, p['W_R'][:, :D].T, p['W_C'][:, :D].T,
            p['W_L'][:, :D].T, w1a_t, w1b_t, p['b_att1'].reshape(1, 64))
        w2 = p['W_att2'][0]
        b2v = jnp.full((16,), p['b_att2'][0], jnp.float32)
        acc = _sc_edge(t8.reshape(8 * N, ROWW), adst_tab, cidx_r, dst_r,
                       attr_r, w2, b2v, zeros)
        bmat = jnp.concatenate([
            p['W_R'][:, D:D + 1].T, p['W_C'][:, D:D + 1].T,
            p['W_L'][:, D:D + 1].T, p['b_R'].reshape(1, D),
            p['b_C'].reshape(1, D), p['b_L'].reshape(1, D),
            jnp.zeros((ACCW - D - 6, D), jnp.float32)], axis=0)
        h = _finalize(acc, bmat, p['bias'].reshape(1, D),
                      p['ln_g'].reshape(1, D), p['ln_b'].reshape(1, D),
                      h, apply_relu=(i < nlayers - 1))
    return h


# speedup vs baseline: 7.2053x; 1.2204x over previous
import functools

import jax
import jax.numpy as jnp
from jax import lax
from jax.experimental import pallas as pl
from jax.experimental.pallas import tpu as pltpu
from jax.experimental.pallas import tpu_sc as plsc

N = 10000
E = 320000
D = 128
NW = 32
EPW = E // NW
NCH = EPW // 16
ACCW = 144
NPAD = 10000
ROWW = 192


def _prep_body(src_ref, vr_ref, vc_ref, vl_ref, cidx_ref):
    is_r = (jnp.abs(vr_ref[...]) > 0.01).astype(jnp.int32)
    is_c = (jnp.abs(vc_ref[...]) > 0.01).astype(jnp.int32)
    is_l = (jnp.abs(vl_ref[...]) > 0.01).astype(jnp.int32)
    combo = is_r + 2 * is_c + 4 * is_l
    cidx_ref[...] = combo * N + src_ref[...]


def _prep(src2, vr2, vc2, vl2):
    return pl.pallas_call(
        _prep_body,
        out_shape=jax.ShapeDtypeStruct(src2.shape, jnp.int32),
    )(src2, vr2, vc2, vl2)


def _tables_body(h_ref, wn_ref, wr_ref, wc_ref, wl_ref, w1a_ref, w1b_ref,
                 b1_ref, t8_ref, adst_ref):
    x_lin = jnp.dot(h_ref[...], wn_ref[...], preferred_element_type=jnp.float32)
    m_r = jnp.dot(x_lin, wr_ref[...], preferred_element_type=jnp.float32)
    m_c = jnp.dot(x_lin, wc_ref[...], preferred_element_type=jnp.float32)
    m_l = jnp.dot(x_lin, wl_ref[...], preferred_element_type=jnp.float32)
    a_src = jnp.dot(x_lin, w1b_ref[...], preferred_element_type=jnp.float32)
    a_dst = jnp.dot(x_lin, w1a_ref[...], preferred_element_type=jnp.float32)
    adst_ref[...] = a_dst + b1_ref[...]
    z = jnp.zeros_like(m_r)
    combos = (z, m_r, m_c, m_r + m_c, m_l, m_r + m_l, m_c + m_l,
              m_r + m_c + m_l)
    t8_ref[...] = jnp.stack(
        [jnp.concatenate([cmb, a_src], axis=1) for cmb in combos], axis=0)


def _tables(h, wn_t, wr_t, wc_t, wl_t, w1a_t, w1b_t, b1):
    blk = 1000
    grid = N // blk
    full = lambda shape: pl.BlockSpec(shape, lambda i: (0,) * len(shape))
    return pl.pallas_call(
        _tables_body,
        grid=(grid,),
        in_specs=[
            pl.BlockSpec((blk, D), lambda i: (i, 0)),
            full((D, D)), full((D, D)), full((D, D)), full((D, D)),
            full((D, 64)), full((D, 64)), full((1, 64)),
        ],
        out_specs=[
            pl.BlockSpec((8, blk, ROWW), lambda i: (0, i, 0)),
            pl.BlockSpec((blk, 64), lambda i: (i, 0)),
        ],
        out_shape=[
            jax.ShapeDtypeStruct((8, N, ROWW), jnp.float32),
            jax.ShapeDtypeStruct((N, 64), jnp.float32),
        ],
    )(h, wn_t, wr_t, wc_t, wl_t, w1a_t, w1b_t, b1)


def _sc_edge_body(t8_hbm, adst_hbm, cidx_hbm, dst_hbm, attr_hbm, w2_hbm,
                  b2_hbm, zeros_hbm, out_hbm,
                  cidx_v, dst_v, w2_v, b2_v, rows0, rows1, rows2,
                  adst0, adst1, adst2, attr0, attr1, attr2,
                  stage0, stage1, stage2, hacc_v, aux_tr, acc_sh,
                  gsem0, gsem1, gsem2, ssem0, ssem1, ssem2):
    c = lax.axis_index("c")
    s = lax.axis_index("s")
    wid = c * 16 + s

    pltpu.sync_copy(cidx_hbm.at[wid], cidx_v)
    pltpu.sync_copy(dst_hbm.at[wid], dst_v)
    pltpu.sync_copy(w2_hbm, w2_v)
    pltpu.sync_copy(b2_hbm, b2_v)

    rpw = NPAD // 16
    pltpu.sync_copy(zeros_hbm.at[pl.ds(s * rpw, rpw)],
                    acc_sh.at[pl.ds(s * rpw, rpw)])

    iota = lax.iota(jnp.int32, 16)
    zero16 = jnp.zeros((16,), jnp.float32)
    for e in range(16):
        aux_tr[pl.ds(e * 16, 16)] = zero16
    plsc.subcore_barrier()

    bufs = ((rows0, adst0, attr0, stage0, gsem0, ssem0),
            (rows1, adst1, attr1, stage1, gsem1, ssem1),
            (rows2, adst2, attr2, stage2, gsem2, ssem2))

    def fire_gathers(k, b):
        rows_v, adst_v, attr_c, _, gsem, _ = bufs[b]
        pltpu.async_copy(t8_hbm.at[cidx_v.at[k]], rows_v, gsem)
        pltpu.async_copy(adst_hbm.at[dst_v.at[k]], adst_v, gsem)
        pltpu.async_copy(attr_hbm.at[wid, k], attr_c, gsem)

    def process(k, b):
        rows_v, adst_v, attr_c, stage_v, gsem, ssem = bufs[b]
        dst_row = dst_v.at[k]
        pltpu.make_async_copy(t8_hbm.at[cidx_v.at[k]], rows_v, gsem).wait()
        pltpu.make_async_copy(adst_hbm.at[dst_v.at[k]], adst_v, gsem).wait()
        pltpu.make_async_copy(attr_hbm.at[wid, k], attr_c, gsem).wait()

        w2q = [w2_v[pl.ds(q * 16, 16)] for q in range(4)]
        for e in range(16):
            acc = zero16
            for q in range(4):
                a_s = rows_v[e, pl.ds(D + q * 16, 16)]
                a_d = adst_v[e, pl.ds(q * 16, 16)]
                hrelu = jnp.maximum(a_s + a_d, 0.0)
                acc = acc + hrelu * w2q[q]
            hacc_v[pl.ds(e * 16, 16)] = acc
        z = b2_v[...]
        for l in range(16):
            z = z + plsc.load_gather(hacc_v, [iota * 16 + l])
        att = 1.0 / (1.0 + jnp.exp(-z))

        v_r = attr_c[pl.ds(0, 16)]
        v_c = attr_c[pl.ds(16, 16)]
        v_l = attr_c[pl.ds(32, 16)]
        a_r = jnp.where(jnp.abs(v_r) > 0.01, att, 0.0)
        a_c = jnp.where(jnp.abs(v_c) > 0.01, att, 0.0)
        a_l = jnp.where(jnp.abs(v_l) > 0.01, att, 0.0)
        for j, vec in enumerate((a_r * v_r, a_c * v_c, a_l * v_l,
                                 a_r, a_c, a_l)):
            plsc.store_scatter(aux_tr, [iota * 16 + j], vec)

        if isinstance(k, int):
            if k >= 3:
                pltpu.make_async_copy(stage_v, acc_sh.at[dst_row], ssem).wait()
        else:

            @pl.when(k >= 3)
            def _():
                pltpu.make_async_copy(stage_v, acc_sh.at[dst_row], ssem).wait()

        for e in range(16):
            att_e = att[e]
            for q in range(8):
                stage_v[e, pl.ds(q * 16, 16)] = (
                    rows_v[e, pl.ds(q * 16, 16)] * att_e)
            stage_v[e, pl.ds(D, 16)] = aux_tr[pl.ds(e * 16, 16)]

        pltpu.async_copy(stage_v, acc_sh.at[dst_row], ssem, add=True)

    fire_gathers(0, 0)
    fire_gathers(1, 1)
    fire_gathers(2, 2)

    def triple(g, carry):
        k0 = g * 3
        for j in range(3):
            process(k0 + j, j)

            @pl.when(k0 + j + 3 < NCH)
            def _():
                fire_gathers(k0 + j + 3, j)

        return carry

    lax.fori_loop(0, (NCH - 1) // 3, triple, 0)
    process(NCH - 1, 0)

    pltpu.make_async_copy(stage1, acc_sh.at[dst_v.at[NCH - 3]], ssem1).wait()
    pltpu.make_async_copy(stage2, acc_sh.at[dst_v.at[NCH - 2]], ssem2).wait()
    pltpu.make_async_copy(stage0, acc_sh.at[dst_v.at[NCH - 1]], ssem0).wait()
    plsc.subcore_barrier()

    pltpu.sync_copy(acc_sh.at[pl.ds(s * rpw, rpw)],
                    out_hbm.at[c, pl.ds(s * rpw, rpw)])


@functools.partial(jax.jit, static_argnums=())
def _sc_edge(t8_flat, adst_tab, cidx_r, dst_r, attr_r, w2, b2v, zeros):
    mesh = plsc.VectorSubcoreMesh(core_axis_name="c", subcore_axis_name="s")
    kern = pl.kernel(
        _sc_edge_body,
        out_type=jax.ShapeDtypeStruct((2, NPAD, ACCW), jnp.float32),
        mesh=mesh,
        compiler_params=pltpu.CompilerParams(use_tc_tiling_on_sc=False,
                                             needs_layout_passes=False),
        scratch_types=[
            pltpu.VMEM((NCH, 16), jnp.int32),
            pltpu.VMEM((NCH, 16), jnp.int32),
            pltpu.VMEM((64,), jnp.float32),
            pltpu.VMEM((16,), jnp.float32),
            pltpu.VMEM((16, ROWW), jnp.float32),
            pltpu.VMEM((16, ROWW), jnp.float32),
            pltpu.VMEM((16, ROWW), jnp.float32),
            pltpu.VMEM((16, 64), jnp.float32),
            pltpu.VMEM((16, 64), jnp.float32),
            pltpu.VMEM((16, 64), jnp.float32),
            pltpu.VMEM((48,), jnp.float32),
            pltpu.VMEM((48,), jnp.float32),
            pltpu.VMEM((48,), jnp.float32),
            pltpu.VMEM((16, ACCW), jnp.float32),
            pltpu.VMEM((16, ACCW), jnp.float32),
            pltpu.VMEM((16, ACCW), jnp.float32),
            pltpu.VMEM((256,), jnp.float32),
            pltpu.VMEM((256,), jnp.float32),
            pltpu.VMEM_SHARED((NPAD, ACCW), jnp.float32),
            pltpu.SemaphoreType.DMA,
            pltpu.SemaphoreType.DMA,
            pltpu.SemaphoreType.DMA,
            pltpu.SemaphoreType.DMA,
            pltpu.SemaphoreType.DMA,
            pltpu.SemaphoreType.DMA,
        ],
    )
    return kern(t8_flat, adst_tab, cidx_r, dst_r, attr_r, w2, b2v, zeros)


def _final_body(acc_ref, bmat_ref, bias_ref, g_ref, b_ref, hin_ref, out_ref,
                *, apply_relu):
    a = acc_ref[0] + acc_ref[1]
    out = a[:, :D] + jnp.dot(a[:, D:ACCW], bmat_ref[...],
                             preferred_element_type=jnp.float32)
    out = out + bias_ref[...]
    mu = jnp.mean(out, axis=1, keepdims=True)
    var = jnp.mean((out - mu) ** 2, axis=1, keepdims=True)
    out = (out - mu) * lax.rsqrt(var + 1e-5) * g_ref[...] + b_ref[...]
    if apply_relu:
        out = jnp.maximum(out, 0.0)
    out_ref[...] = out + hin_ref[...]


def _finalize(acc, bmat, bias, g, b, h_in, apply_relu):
    blk = 1000
    grid = N // blk
    full = lambda shape: pl.BlockSpec(shape, lambda i: (0,) * len(shape))
    return pl.pallas_call(
        functools.partial(_final_body, apply_relu=apply_relu),
        grid=(grid,),
        in_specs=[
            pl.BlockSpec((2, blk, ACCW), lambda i: (0, i, 0)),
            full((ACCW - D, D)), full((1, D)), full((1, D)), full((1, D)),
            pl.BlockSpec((blk, D), lambda i: (i, 0)),
        ],
        out_specs=pl.BlockSpec((blk, D), lambda i: (i, 0)),
        out_shape=jax.ShapeDtypeStruct((N, D), jnp.float32),
    )(acc, bmat, bias, g, b, h_in)


def kernel(x, edge_index, edge_attr, params):
    src = edge_index[0]
    dst = edge_index[1]
    vr = edge_attr[:, 0]
    vc = edge_attr[:, 1]
    vl = edge_attr[:, 2]

    cidx = _prep(src.reshape(2500, 128), vr.reshape(2500, 128),
                 vc.reshape(2500, 128), vl.reshape(2500, 128))
    cidx_r = cidx.reshape(NW, NCH, 16)
    dst_r = dst.reshape(NW, NCH, 16)
    attr_r = jnp.transpose(edge_attr.reshape(NW, NCH, 16, 3),
                           (0, 1, 3, 2)).reshape(NW, NCH, 48)
    zeros = jnp.zeros((NPAD, ACCW), jnp.float32)

    h = x
    nlayers = len(params)
    for i, p in enumerate(params):
        w1a_t = p['W_att1'][:, :D].T
        w1b_t = p['W_att1'][:, D:].T
        t8, adst_tab = _tables(
            h, p['W_node'].T, p['W_R'][:, :D].T, p['W_C'][:, :D].T,
            p['W_L'][:, :D].T, w1a_t, w1b_t, p['b_att1'].reshape(1, 64))
        w2 = p['W_att2'][0]
        b2v = jnp.full((16,), p['b_att2'][0], jnp.float32)
        acc = _sc_edge(t8.reshape(8 * N, ROWW), adst_tab, cidx_r, dst_r,
                       attr_r, w2, b2v, zeros)
        bmat = jnp.concatenate([
            p['W_R'][:, D:D + 1].T, p['W_C'][:, D:D + 1].T,
            p['W_L'][:, D:D + 1].T, p['b_R'].reshape(1, D),
            p['b_C'].reshape(1, D), p['b_L'].reshape(1, D),
            jnp.zeros((ACCW - D - 6, D), jnp.float32)], axis=0)
        h = _finalize(acc, bmat, p['bias'].reshape(1, D),
                      p['ln_g'].reshape(1, D), p['ln_b'].reshape(1, D),
                      h, apply_relu=(i < nlayers - 1))
    return h


# speedup vs baseline: 7.2692x; 1.0089x over previous
import functools

import jax
import jax.numpy as jnp
from jax import lax
from jax.experimental import pallas as pl
from jax.experimental.pallas import tpu as pltpu
from jax.experimental.pallas import tpu_sc as plsc

N = 10000
E = 320000
D = 128
NW = 32
EPW = E // NW
NCH = EPW // 16
ACCW = 144
NPAD = 10000
ROWW = 192


def _prep_body(src_ref, vr_ref, vc_ref, vl_ref, cidx_ref):
    is_r = (jnp.abs(vr_ref[...]) > 0.01).astype(jnp.int32)
    is_c = (jnp.abs(vc_ref[...]) > 0.01).astype(jnp.int32)
    is_l = (jnp.abs(vl_ref[...]) > 0.01).astype(jnp.int32)
    combo = is_r + 2 * is_c + 4 * is_l
    cidx_ref[...] = combo * N + src_ref[...]


def _prep(src2, vr2, vc2, vl2):
    return pl.pallas_call(
        _prep_body,
        out_shape=jax.ShapeDtypeStruct(src2.shape, jnp.int32),
    )(src2, vr2, vc2, vl2)


def _tables_body(h_ref, wn_ref, wr_ref, wc_ref, wl_ref, w1a_ref, w1b_ref,
                 b1_ref, t8_ref, adst_ref):
    x_lin = jnp.dot(h_ref[...], wn_ref[...], preferred_element_type=jnp.float32)
    m_r = jnp.dot(x_lin, wr_ref[...], preferred_element_type=jnp.float32)
    m_c = jnp.dot(x_lin, wc_ref[...], preferred_element_type=jnp.float32)
    m_l = jnp.dot(x_lin, wl_ref[...], preferred_element_type=jnp.float32)
    a_src = jnp.dot(x_lin, w1b_ref[...], preferred_element_type=jnp.float32)
    a_dst = jnp.dot(x_lin, w1a_ref[...], preferred_element_type=jnp.float32)
    adst_ref[...] = a_dst + b1_ref[...]
    z = jnp.zeros_like(m_r)
    combos = (z, m_r, m_c, m_r + m_c, m_l, m_r + m_l, m_c + m_l,
              m_r + m_c + m_l)
    t8_ref[...] = jnp.stack(
        [jnp.concatenate([cmb, a_src], axis=1) for cmb in combos], axis=0)


def _tables(h, wn_t, wr_t, wc_t, wl_t, w1a_t, w1b_t, b1):
    blk = 1000
    grid = N // blk
    full = lambda shape: pl.BlockSpec(shape, lambda i: (0,) * len(shape))
    return pl.pallas_call(
        _tables_body,
        grid=(grid,),
        in_specs=[
            pl.BlockSpec((blk, D), lambda i: (i, 0)),
            full((D, D)), full((D, D)), full((D, D)), full((D, D)),
            full((D, 64)), full((D, 64)), full((1, 64)),
        ],
        out_specs=[
            pl.BlockSpec((8, blk, ROWW), lambda i: (0, i, 0)),
            pl.BlockSpec((blk, 64), lambda i: (i, 0)),
        ],
        out_shape=[
            jax.ShapeDtypeStruct((8, N, ROWW), jnp.float32),
            jax.ShapeDtypeStruct((N, 64), jnp.float32),
        ],
    )(h, wn_t, wr_t, wc_t, wl_t, w1a_t, w1b_t, b1)


def _sc_edge_body(t8_hbm, adst_hbm, cidx_hbm, dst_hbm, attr_hbm, w2_hbm,
                  b2_hbm, zeros_hbm, out_hbm,
                  cidx_v, dst_v, w2_v, b2_v, rows0, rows1, rows2,
                  adst0, adst1, adst2, attr0, attr1, attr2,
                  stage0, stage1, stage2, hacc_v, aux_tr, acc_sh,
                  gsem0, gsem1, gsem2, ssem0, ssem1, ssem2):
    c = lax.axis_index("c")
    s = lax.axis_index("s")
    wid = c * 16 + s

    pltpu.sync_copy(cidx_hbm.at[wid], cidx_v)
    pltpu.sync_copy(dst_hbm.at[wid], dst_v)
    pltpu.sync_copy(w2_hbm, w2_v)
    pltpu.sync_copy(b2_hbm, b2_v)

    rpw = NPAD // 16
    pltpu.sync_copy(zeros_hbm.at[pl.ds(s * rpw, rpw)],
                    acc_sh.at[pl.ds(s * rpw, rpw)])

    iota = lax.iota(jnp.int32, 16)
    zero16 = jnp.zeros((16,), jnp.float32)
    for e in range(16):
        aux_tr[pl.ds(e * 16, 16)] = zero16
    plsc.subcore_barrier()

    bufs = ((rows0, adst0, attr0, stage0, gsem0, ssem0),
            (rows1, adst1, attr1, stage1, gsem1, ssem1),
            (rows2, adst2, attr2, stage2, gsem2, ssem2))

    def fire_gathers(k, b):
        rows_v, adst_v, attr_c, _, gsem, _ = bufs[b]
        pltpu.async_copy(t8_hbm.at[cidx_v.at[k]], rows_v, gsem)
        pltpu.async_copy(adst_hbm.at[dst_v.at[k]], adst_v, gsem)
        pltpu.async_copy(attr_hbm.at[wid, k], attr_c, gsem)

    def process(k, b):
        rows_v, adst_v, attr_c, stage_v, gsem, ssem = bufs[b]
        dst_row = dst_v.at[k]
        pltpu.make_async_copy(t8_hbm.at[cidx_v.at[k]], rows_v, gsem).wait()
        pltpu.make_async_copy(adst_hbm.at[dst_v.at[k]], adst_v, gsem).wait()
        pltpu.make_async_copy(attr_hbm.at[wid, k], attr_c, gsem).wait()

        w2q = [w2_v[pl.ds(q * 16, 16)] for q in range(4)]
        for e in range(16):
            acc = zero16
            for q in range(4):
                a_s = rows_v[e, pl.ds(D + q * 16, 16)]
                a_d = adst_v[e, pl.ds(q * 16, 16)]
                hrelu = jnp.maximum(a_s + a_d, 0.0)
                acc = acc + hrelu * w2q[q]
            hacc_v[pl.ds(e * 16, 16)] = acc
        z = b2_v[...]
        for l in range(16):
            z = z + plsc.load_gather(hacc_v, [iota * 16 + l])
        att = 1.0 / (1.0 + jnp.exp(-z))

        v_r = attr_c[pl.ds(0, 16)]
        v_c = attr_c[pl.ds(16, 16)]
        v_l = attr_c[pl.ds(32, 16)]
        a_r = jnp.where(jnp.abs(v_r) > 0.01, att, 0.0)
        a_c = jnp.where(jnp.abs(v_c) > 0.01, att, 0.0)
        a_l = jnp.where(jnp.abs(v_l) > 0.01, att, 0.0)
        for j, vec in enumerate((a_r * v_r, a_c * v_c, a_l * v_l,
                                 a_r, a_c, a_l)):
            plsc.store_scatter(aux_tr, [iota * 16 + j], vec)

        if isinstance(k, int):
            if k >= 3:
                pltpu.make_async_copy(stage_v, acc_sh.at[dst_row], ssem).wait()
        else:

            @pl.when(k >= 3)
            def _():
                pltpu.make_async_copy(stage_v, acc_sh.at[dst_row], ssem).wait()

        for e in range(16):
            att_e = att[e]
            for q in range(8):
                stage_v[e, pl.ds(q * 16, 16)] = (
                    rows_v[e, pl.ds(q * 16, 16)] * att_e)
            stage_v[e, pl.ds(D, 16)] = aux_tr[pl.ds(e * 16, 16)]

        pltpu.async_copy(stage_v, acc_sh.at[dst_row], ssem, add=True)

    fire_gathers(0, 0)
    fire_gathers(1, 1)
    fire_gathers(2, 2)

    def triple(g, carry):
        k0 = g * 3
        for j in range(3):
            process(k0 + j, j)

            @pl.when(k0 + j + 3 < NCH)
            def _():
                fire_gathers(k0 + j + 3, j)

        return carry

    lax.fori_loop(0, (NCH - 1) // 3, triple, 0)
    process(NCH - 1, 0)

    pltpu.make_async_copy(stage1, acc_sh.at[dst_v.at[NCH - 3]], ssem1).wait()
    pltpu.make_async_copy(stage2, acc_sh.at[dst_v.at[NCH - 2]], ssem2).wait()
    pltpu.make_async_copy(stage0, acc_sh.at[dst_v.at[NCH - 1]], ssem0).wait()
    plsc.subcore_barrier()

    pltpu.sync_copy(acc_sh.at[pl.ds(s * rpw, rpw)],
                    out_hbm.at[c, pl.ds(s * rpw, rpw)])


@functools.partial(jax.jit, static_argnums=())
def _sc_edge(t8_flat, adst_tab, cidx_r, dst_r, attr_r, w2, b2v, zeros):
    mesh = plsc.VectorSubcoreMesh(core_axis_name="c", subcore_axis_name="s")
    kern = pl.kernel(
        _sc_edge_body,
        out_type=jax.ShapeDtypeStruct((2, NPAD, ACCW), jnp.float32),
        mesh=mesh,
        compiler_params=pltpu.CompilerParams(use_tc_tiling_on_sc=False,
                                             needs_layout_passes=False),
        scratch_types=[
            pltpu.VMEM((NCH, 16), jnp.int32),
            pltpu.VMEM((NCH, 16), jnp.int32),
            pltpu.VMEM((64,), jnp.float32),
            pltpu.VMEM((16,), jnp.float32),
            pltpu.VMEM((16, ROWW), jnp.float32),
            pltpu.VMEM((16, ROWW), jnp.float32),
            pltpu.VMEM((16, ROWW), jnp.float32),
            pltpu.VMEM((16, 64), jnp.float32),
            pltpu.VMEM((16, 64), jnp.float32),
            pltpu.VMEM((16, 64), jnp.float32),
            pltpu.VMEM((48,), jnp.float32),
            pltpu.VMEM((48,), jnp.float32),
            pltpu.VMEM((48,), jnp.float32),
            pltpu.VMEM((16, ACCW), jnp.float32),
            pltpu.VMEM((16, ACCW), jnp.float32),
            pltpu.VMEM((16, ACCW), jnp.float32),
            pltpu.VMEM((256,), jnp.float32),
            pltpu.VMEM((256,), jnp.float32),
            pltpu.VMEM_SHARED((NPAD, ACCW), jnp.float32),
            pltpu.SemaphoreType.DMA,
            pltpu.SemaphoreType.DMA,
            pltpu.SemaphoreType.DMA,
            pltpu.SemaphoreType.DMA,
            pltpu.SemaphoreType.DMA,
            pltpu.SemaphoreType.DMA,
        ],
    )
    return kern(t8_flat, adst_tab, cidx_r, dst_r, attr_r, w2, b2v, zeros)


def _fused_body(acc_ref, bmat_ref, bias_ref, g_ref, b_ref, hin_ref,
                wn_ref, wr_ref, wc_ref, wl_ref, w1a_ref, w1b_ref, b1_ref,
                h_ref, t8_ref, adst_ref, *, apply_relu):
    a = acc_ref[0] + acc_ref[1]
    out = a[:, :D] + jnp.dot(a[:, D:ACCW], bmat_ref[...],
                             preferred_element_type=jnp.float32)
    out = out + bias_ref[...]
    mu = jnp.mean(out, axis=1, keepdims=True)
    var = jnp.mean((out - mu) ** 2, axis=1, keepdims=True)
    out = (out - mu) * lax.rsqrt(var + 1e-5) * g_ref[...] + b_ref[...]
    if apply_relu:
        out = jnp.maximum(out, 0.0)
    h = out + hin_ref[...]
    h_ref[...] = h
    x_lin = jnp.dot(h, wn_ref[...], preferred_element_type=jnp.float32)
    m_r = jnp.dot(x_lin, wr_ref[...], preferred_element_type=jnp.float32)
    m_c = jnp.dot(x_lin, wc_ref[...], preferred_element_type=jnp.float32)
    m_l = jnp.dot(x_lin, wl_ref[...], preferred_element_type=jnp.float32)
    a_src = jnp.dot(x_lin, w1b_ref[...], preferred_element_type=jnp.float32)
    a_dst = jnp.dot(x_lin, w1a_ref[...], preferred_element_type=jnp.float32)
    adst_ref[...] = a_dst + b1_ref[...]
    z = jnp.zeros_like(m_r)
    combos = (z, m_r, m_c, m_r + m_c, m_l, m_r + m_l, m_c + m_l,
              m_r + m_c + m_l)
    t8_ref[...] = jnp.stack(
        [jnp.concatenate([cmb, a_src], axis=1) for cmb in combos], axis=0)


def _fused(acc, bmat, bias, g, b, h_in, wn_t, wr_t, wc_t, wl_t, w1a_t,
           w1b_t, b1, apply_relu):
    blk = 1000
    grid = N // blk
    full = lambda shape: pl.BlockSpec(shape, lambda i: (0,) * len(shape))
    return pl.pallas_call(
        functools.partial(_fused_body, apply_relu=apply_relu),
        grid=(grid,),
        in_specs=[
            pl.BlockSpec((2, blk, ACCW), lambda i: (0, i, 0)),
            full((ACCW - D, D)), full((1, D)), full((1, D)), full((1, D)),
            pl.BlockSpec((blk, D), lambda i: (i, 0)),
            full((D, D)), full((D, D)), full((D, D)), full((D, D)),
            full((D, 64)), full((D, 64)), full((1, 64)),
        ],
        out_specs=[
            pl.BlockSpec((blk, D), lambda i: (i, 0)),
            pl.BlockSpec((8, blk, ROWW), lambda i: (0, i, 0)),
            pl.BlockSpec((blk, 64), lambda i: (i, 0)),
        ],
        out_shape=[
            jax.ShapeDtypeStruct((N, D), jnp.float32),
            jax.ShapeDtypeStruct((8, N, ROWW), jnp.float32),
            jax.ShapeDtypeStruct((N, 64), jnp.float32),
        ],
    )(acc, bmat, bias, g, b, h_in, wn_t, wr_t, wc_t, wl_t, w1a_t, w1b_t, b1)


def _final_body(acc_ref, bmat_ref, bias_ref, g_ref, b_ref, hin_ref, out_ref,
                *, apply_relu):
    a = acc_ref[0] + acc_ref[1]
    out = a[:, :D] + jnp.dot(a[:, D:ACCW], bmat_ref[...],
                             preferred_element_type=jnp.float32)
    out = out + bias_ref[...]
    mu = jnp.mean(out, axis=1, keepdims=True)
    var = jnp.mean((out - mu) ** 2, axis=1, keepdims=True)
    out = (out - mu) * lax.rsqrt(var + 1e-5) * g_ref[...] + b_ref[...]
    if apply_relu:
        out = jnp.maximum(out, 0.0)
    out_ref[...] = out + hin_ref[...]


def _finalize(acc, bmat, bias, g, b, h_in, apply_relu):
    blk = 1000
    grid = N // blk
    full = lambda shape: pl.BlockSpec(shape, lambda i: (0,) * len(shape))
    return pl.pallas_call(
        functools.partial(_final_body, apply_relu=apply_relu),
        grid=(grid,),
        in_specs=[
            pl.BlockSpec((2, blk, ACCW), lambda i: (0, i, 0)),
            full((ACCW - D, D)), full((1, D)), full((1, D)), full((1, D)),
            pl.BlockSpec((blk, D), lambda i: (i, 0)),
        ],
        out_specs=pl.BlockSpec((blk, D), lambda i: (i, 0)),
        out_shape=jax.ShapeDtypeStruct((N, D), jnp.float32),
    )(acc, bmat, bias, g, b, h_in)


def kernel(x, edge_index, edge_attr, params):
    src = edge_index[0]
    dst = edge_index[1]
    vr = edge_attr[:, 0]
    vc = edge_attr[:, 1]
    vl = edge_attr[:, 2]

    cidx = _prep(src.reshape(2500, 128), vr.reshape(2500, 128),
                 vc.reshape(2500, 128), vl.reshape(2500, 128))
    cidx_r = cidx.reshape(NW, NCH, 16)
    dst_r = dst.reshape(NW, NCH, 16)
    attr_r = jnp.transpose(edge_attr.reshape(NW, NCH, 16, 3),
                           (0, 1, 3, 2)).reshape(NW, NCH, 48)
    zeros = jnp.zeros((NPAD, ACCW), jnp.float32)

    def tabw(p):
        return (p['W_node'].T, p['W_R'][:, :D].T, p['W_C'][:, :D].T,
                p['W_L'][:, :D].T, p['W_att1'][:, :D].T,
                p['W_att1'][:, D:].T, p['b_att1'].reshape(1, 64))

    def bmat_of(p):
        return jnp.concatenate([
            p['W_R'][:, D:D + 1].T, p['W_C'][:, D:D + 1].T,
            p['W_L'][:, D:D + 1].T, p['b_R'].reshape(1, D),
            p['b_C'].reshape(1, D), p['b_L'].reshape(1, D),
            jnp.zeros((ACCW - D - 6, D), jnp.float32)], axis=0)

    h = x
    nlayers = len(params)
    t8, adst_tab = _tables(h, *tabw(params[0]))
    for i, p in enumerate(params):
        w2 = p['W_att2'][0]
        b2v = jnp.full((16,), p['b_att2'][0], jnp.float32)
        acc = _sc_edge(t8.reshape(8 * N, ROWW), adst_tab, cidx_r, dst_r,
                       attr_r, w2, b2v, zeros)
        lnorm = (bmat_of(p), p['bias'].reshape(1, D),
                 p['ln_g'].reshape(1, D), p['ln_b'].reshape(1, D))
        if i < nlayers - 1:
            h, t8, adst_tab = _fused(acc, *lnorm, h, *tabw(params[i + 1]),
                                     apply_relu=True)
        else:
            h = _finalize(acc, *lnorm, h, apply_relu=False)
    return h
